# baseline (speedup 1.0000x reference)
import functools

import jax
import jax.numpy as jnp
from jax import lax
from jax.experimental import pallas as pl
from jax.experimental.pallas import tpu as pltpu
from jax.experimental.pallas import tpu_sc as plsc

_N = 10000
_E = 160000
_F = 512
_H1 = 32
_H2 = 16

_NS = 16
_NW = _NS
_CH = 128
_EPAD = 163840
_EPW = _EPAD // _NW
_G = _EPW // _CH
_RPT = _N // _NS


def _spmm_sc(support, src3, dst3, ew3, zeros_tile):
    mesh = plsc.VectorSubcoreMesh(core_axis_name="c", subcore_axis_name="s",
                                  num_cores=1)

    @functools.partial(
        pl.kernel,
        out_type=jax.ShapeDtypeStruct((_NS, _RPT, _H1), jnp.float32),
        mesh=mesh,
        scratch_types=[
            pltpu.VMEM((_G, _CH), jnp.int32),
            pltpu.VMEM((_G, _CH), jnp.int32),
            pltpu.VMEM((_G, _CH), jnp.float32),
            pltpu.VMEM((_CH, _H1), jnp.float32),
            pltpu.VMEM((_CH, _H1), jnp.float32),
            pltpu.VMEM((_RPT, _H1), jnp.float32),
            pltpu.VMEM_SHARED((_N, _H1), jnp.float32),
            pltpu.SemaphoreType.DMA,
            pltpu.SemaphoreType.DMA,
        ],
        compiler_params=pltpu.CompilerParams(use_tc_tiling_on_sc=False),
    )
    def k(support_hbm, src_hbm, dst_hbm, ew_hbm, zt_hbm, out_hbm,
          src_v, dst_v, ew_v, rows_a, rows_b, stage_v, acc, sem_a, sem_b):
        s = lax.axis_index("s")
        wid = s
        pltpu.sync_copy(zt_hbm, stage_v)
        pltpu.sync_copy(stage_v, acc.at[pl.ds(s * _RPT, _RPT)])
        pltpu.sync_copy(src_hbm.at[wid], src_v)
        pltpu.sync_copy(dst_hbm.at[wid], dst_v)
        pltpu.sync_copy(ew_hbm.at[wid], ew_v)
        plsc.subcore_barrier()

        def scale_scatter(g, rows_v):
            def grp(b, c2):
                base = b * 16
                ew16 = ew_v[g, pl.ds(base, 16)]
                for j in range(16):
                    w = ew16[j]
                    e = base + j
                    rows_v[e, pl.ds(0, 16)] = rows_v[e, pl.ds(0, 16)] * w
                    rows_v[e, pl.ds(16, 16)] = rows_v[e, pl.ds(16, 16)] * w
                return c2

            lax.fori_loop(0, _CH // 16, grp, 0)
            pltpu.sync_copy(rows_v, acc.at[dst_v.at[g]], add=True)

        pltpu.async_copy(support_hbm.at[src_v.at[0]], rows_a, sem_a)

        def pair(p, carry):
            g0 = 2 * p
            pltpu.async_copy(support_hbm.at[src_v.at[g0 + 1]], rows_b, sem_b)
            pltpu.make_async_copy(support_hbm.at[src_v.at[g0]], rows_a,
                                  sem_a).wait()
            scale_scatter(g0, rows_a)

            @pl.when(p < _G // 2 - 1)
            def _():
                pltpu.async_copy(support_hbm.at[src_v.at[g0 + 2]], rows_a,
                                 sem_a)

            pltpu.make_async_copy(support_hbm.at[src_v.at[g0 + 1]], rows_b,
                                  sem_b).wait()
            scale_scatter(g0 + 1, rows_b)
            return carry

        lax.fori_loop(0, _G // 2, pair, 0)
        plsc.subcore_barrier()
        pltpu.sync_copy(acc.at[pl.ds(s * _RPT, _RPT)], stage_v)
        pltpu.sync_copy(stage_v, out_hbm.at[s])

    return k(support, src3, dst3, ew3, zeros_tile).reshape(_N, _H1)


_TMA = 2000


def _support1_tc(x, w1):
    def body(x_ref, w_ref, o_ref):
        o_ref[...] = lax.dot_general(
            x_ref[...], w_ref[...], (((1,), (0,)), ((), ())),
            preferred_element_type=jnp.float32)

    return pl.pallas_call(
        body,
        grid=(_N // _TMA,),
        in_specs=[pl.BlockSpec((_TMA, _F), lambda i: (i, 0)),
                  pl.BlockSpec((_F, _H1), lambda i: (0, 0))],
        out_specs=pl.BlockSpec((_TMA, _H1), lambda i: (i, 0)),
        out_shape=jax.ShapeDtypeStruct((_N, _H1), jnp.float32),
    )(x, w1)


def _layer2_tc(agg, wcat):
    def body(p_ref, w_ref, o_ref):
        feat = jnp.maximum(p_ref[...], 0.0)
        o_ref[...] = lax.dot_general(
            feat, w_ref[...], (((1,), (0,)), ((), ())),
            preferred_element_type=jnp.float32)

    nb = _N // _TMA
    return pl.pallas_call(
        body,
        grid=(nb,),
        in_specs=[pl.BlockSpec((_TMA, _H1), lambda i: (i, 0)),
                  pl.BlockSpec((_H1, _H1), lambda i: (0, 0))],
        out_specs=pl.BlockSpec((_TMA, _H1), lambda i: (i, 0)),
        out_shape=jax.ShapeDtypeStruct((_N, _H1), jnp.float32),
    )(agg, wcat)


def _z_kl_tc(agg2, eps2):
    def body(p_ref, e_ref, z_ref, kl_ref):
        i = pl.program_id(0)
        both = p_ref[...]
        m = both[:, :_H2]
        ls = both[:, _H2:]
        z_ref[...] = m + jnp.exp(ls) * e_ref[...]
        klp = jnp.sum(1.0 + 2.0 * ls - m * m - jnp.exp(2.0 * ls))

        @pl.when(i == 0)
        def _():
            kl_ref[0, 0] = 0.0

        kl_ref[0, 0] += klp

    nb = _N // _TMA
    return pl.pallas_call(
        body,
        grid=(nb,),
        in_specs=[pl.BlockSpec((_TMA, _H1), lambda i: (i, 0)),
                  pl.BlockSpec((_TMA, _H2), lambda i: (i, 0))],
        out_specs=[pl.BlockSpec((_TMA, _H2), lambda i: (i, 0)),
                   pl.BlockSpec(memory_space=pltpu.SMEM)],
        out_shape=[jax.ShapeDtypeStruct((_N, _H2), jnp.float32),
                   jax.ShapeDtypeStruct((1, 1), jnp.float32)],
    )(agg2, eps2)


_TMD = 200
_THRESH = 0.0016


def _decoder_tc(raw, z, kl_sum):
    nsteps = _N // _TMD

    def body(raw_ref, zr_ref, zf_ref, kl_ref, loss_ref,
             lab_acc, s1_acc, s2_acc):
        i = pl.program_id(0)

        @pl.when(i == 0)
        def _():
            lab_acc[0, 0] = 0.0
            s1_acc[0, 0] = 0.0
            s2_acc[0, 0] = 0.0

        hat = lax.dot_general(
            zr_ref[...], zf_ref[...], (((1,), (1,)), ((), ())),
            preferred_element_type=jnp.float32)
        lab = raw_ref[...] < _THRESH
        t = jnp.exp(-jnp.abs(hat))
        a = jnp.log(1.0 + t) + jnp.maximum(-hat, 0.0)
        s1_acc[0, 0] += jnp.sum(jnp.where(lab, a, 0.0))
        s2_acc[0, 0] += jnp.sum(jnp.where(lab, 0.0, a + hat))
        lab_acc[0, 0] += jnp.sum(jnp.where(lab, 1.0, 0.0))

        @pl.when(i == nsteps - 1)
        def _():
            adj_sum = lab_acc[0, 0]
            s1 = s1_acc[0, 0]
            s2 = s2_acc[0, 0]
            n2 = float(_N) * float(_N)
            norm = n2 / (2.0 * (n2 - adj_sum))
            pos_weight = (n2 - adj_sum) / adj_sum
            recon = norm * (pos_weight * s1 + s2) / n2
            kl = -0.5 * kl_ref[0, 0] / n2
            loss_ref[0, 0] = recon + kl

    return pl.pallas_call(
        body,
        grid=(nsteps,),
        in_specs=[pl.BlockSpec((_TMD, _N), lambda i: (i, 0)),
                  pl.BlockSpec((_TMD, _H2), lambda i: (i, 0)),
                  pl.BlockSpec((_N, _H2), lambda i: (0, 0)),
                  pl.BlockSpec(memory_space=pltpu.SMEM)],
        out_specs=pl.BlockSpec(memory_space=pltpu.SMEM),
        out_shape=jax.ShapeDtypeStruct((1, 1), jnp.float32),
        scratch_shapes=[pltpu.SMEM((1, 1), jnp.float32),
                        pltpu.SMEM((1, 1), jnp.float32),
                        pltpu.SMEM((1, 1), jnp.float32)],
    )(raw, z, z, kl_sum)


def kernel(features_norm, edge_index, edge_weight, eps, adj_label_raw,
           W1, W_mu, W_sigma):
    x = jnp.squeeze(features_norm, 0)
    eps2 = jnp.squeeze(eps, 0)

    pad = _EPAD - _E
    src3 = jnp.concatenate(
        [edge_index[0], jnp.zeros((pad,), jnp.int32)]).reshape(_NW, _G, _CH)
    dst3 = jnp.concatenate(
        [edge_index[1], jnp.zeros((pad,), jnp.int32)]).reshape(_NW, _G, _CH)
    ew3 = jnp.concatenate(
        [edge_weight, jnp.zeros((pad,), jnp.float32)]).reshape(_NW, _G, _CH)
    zeros_tile = jnp.zeros((_RPT, _H1), jnp.float32)

    support1 = _support1_tc(x, W1)
    agg1 = _spmm_sc(support1, src3, dst3, ew3, zeros_tile)
    sup2 = _layer2_tc(agg1, jnp.concatenate([W_mu, W_sigma], axis=1))
    agg2 = _spmm_sc(sup2, src3, dst3, ew3, zeros_tile)
    z, kl_sum = _z_kl_tc(agg2, eps2)
    loss = _decoder_tc(adj_label_raw, z, kl_sum)
    return loss[0, 0]


# speedup vs baseline: 1.0920x; 1.0920x over previous
import functools

import jax
import jax.numpy as jnp
from jax import lax
from jax.experimental import pallas as pl
from jax.experimental.pallas import tpu as pltpu
from jax.experimental.pallas import tpu_sc as plsc

_N = 10000
_E = 160000
_F = 512
_H1 = 32
_H2 = 16

_NC = 2
_NS = 16
_NW = _NC * _NS
_CH = 128
_EPW = 5120
_G = _EPW // _CH
_EPAD = _NW * _EPW
_RPT = _N // _NS


def _spmm_sc(support, src3, dst3, ew3, zeros_tile):
    mesh = plsc.VectorSubcoreMesh(core_axis_name="c", subcore_axis_name="s")

    @functools.partial(
        pl.kernel,
        out_type=jax.ShapeDtypeStruct((_NC, _NS, _RPT, _H1), jnp.float32),
        mesh=mesh,
        scratch_types=[
            pltpu.VMEM((_G, _CH), jnp.int32),
            pltpu.VMEM((_G, _CH), jnp.int32),
            pltpu.VMEM((_G, _CH), jnp.float32),
            pltpu.VMEM((_CH, _H1), jnp.float32),
            pltpu.VMEM((_CH, _H1), jnp.float32),
            pltpu.VMEM((_CH, _H1), jnp.float32),
            pltpu.VMEM((_CH, _H1), jnp.float32),
            pltpu.VMEM((_RPT, _H1), jnp.float32),
            pltpu.VMEM_SHARED((_N, _H1), jnp.float32),
            pltpu.SemaphoreType.DMA,
            pltpu.SemaphoreType.DMA,
            pltpu.SemaphoreType.DMA,
            pltpu.SemaphoreType.DMA,
            pltpu.SemaphoreType.DMA,
            pltpu.SemaphoreType.DMA,
            pltpu.SemaphoreType.DMA,
            pltpu.SemaphoreType.DMA,
        ],
        compiler_params=pltpu.CompilerParams(use_tc_tiling_on_sc=False),
    )
    def k(support_hbm, src_hbm, dst_hbm, ew_hbm, zt_hbm, out_hbm,
          src_v, dst_v, ew_v, r0, r1, r2, r3, stage_v, acc,
          g0, g1, g2, g3, s0, s1, s2, s3):
        c = lax.axis_index("c")
        s = lax.axis_index("s")
        wid = s * _NC + c
        pltpu.sync_copy(zt_hbm, stage_v)
        pltpu.sync_copy(stage_v, acc.at[pl.ds(s * _RPT, _RPT)])
        pltpu.sync_copy(src_hbm.at[wid], src_v)
        pltpu.sync_copy(dst_hbm.at[wid], dst_v)
        pltpu.sync_copy(ew_hbm.at[wid], ew_v)
        plsc.subcore_barrier()

        rows = (r0, r1, r2, r3)
        gsem = (g0, g1, g2, g3)
        ssem = (s0, s1, s2, s3)

        def scale(g, rows_v):
            def grp(b, c2):
                base = b * 16
                ew16 = ew_v[g, pl.ds(base, 16)]
                for j in range(16):
                    w = ew16[j]
                    e = base + j
                    rows_v[e, pl.ds(0, 16)] = rows_v[e, pl.ds(0, 16)] * w
                    rows_v[e, pl.ds(16, 16)] = rows_v[e, pl.ds(16, 16)] * w
                return c2

            lax.fori_loop(0, _CH // 16, grp, 0)

        def drain_scatter(k):
            pltpu.make_async_copy(rows[k], acc.at[dst_v.at[0]],
                                  ssem[k]).wait()

        pltpu.async_copy(support_hbm.at[src_v.at[0]], rows[0], gsem[0])
        pltpu.async_copy(support_hbm.at[src_v.at[1]], rows[1], gsem[1])

        def quad(q, carry):
            for kk in range(4):
                g = 4 * q + kk
                k = kk
                kp = (kk + 2) % 4
                if kk < 2:
                    @pl.when(q > 0)
                    def _():
                        drain_scatter(kp)
                    pltpu.async_copy(support_hbm.at[src_v.at[g + 2]],
                                     rows[kp], gsem[kp])
                else:
                    drain_scatter(kp)

                    @pl.when(q < _G // 4 - 1)
                    def _():
                        pltpu.async_copy(support_hbm.at[src_v.at[g + 2]],
                                         rows[kp], gsem[kp])

                pltpu.make_async_copy(support_hbm.at[src_v.at[0]], rows[k],
                                      gsem[k]).wait()
                scale(g, rows[k])
                pltpu.async_copy(rows[k], acc.at[dst_v.at[g]], ssem[k],
                                 add=True)
            return carry

        lax.fori_loop(0, _G // 4, quad, 0)
        drain_scatter(2)
        drain_scatter(3)
        plsc.subcore_barrier()
        pltpu.sync_copy(acc.at[pl.ds(s * _RPT, _RPT)], stage_v)
        pltpu.sync_copy(stage_v, out_hbm.at[c, s])

    return k(support, src3, dst3, ew3, zeros_tile).reshape(_NC * _N, _H1)


_TMA = 2000


def _support1_tc(x, w1):
    def body(x_ref, w_ref, o_ref):
        o_ref[...] = lax.dot_general(
            x_ref[...], w_ref[...], (((1,), (0,)), ((), ())),
            preferred_element_type=jnp.float32)

    return pl.pallas_call(
        body,
        grid=(_N // _TMA,),
        in_specs=[pl.BlockSpec((_TMA, _F), lambda i: (i, 0)),
                  pl.BlockSpec((_F, _H1), lambda i: (0, 0))],
        out_specs=pl.BlockSpec((_TMA, _H1), lambda i: (i, 0)),
        out_shape=jax.ShapeDtypeStruct((_N, _H1), jnp.float32),
    )(x, w1)


def _layer2_tc(agg, wcat):
    def body(p0_ref, p1_ref, w_ref, o_ref):
        feat = jnp.maximum(p0_ref[...] + p1_ref[...], 0.0)
        o_ref[...] = lax.dot_general(
            feat, w_ref[...], (((1,), (0,)), ((), ())),
            preferred_element_type=jnp.float32)

    nb = _N // _TMA
    return pl.pallas_call(
        body,
        grid=(nb,),
        in_specs=[pl.BlockSpec((_TMA, _H1), lambda i: (i, 0)),
                  pl.BlockSpec((_TMA, _H1), lambda i: (i + nb, 0)),
                  pl.BlockSpec((_H1, _H1), lambda i: (0, 0))],
        out_specs=pl.BlockSpec((_TMA, _H1), lambda i: (i, 0)),
        out_shape=jax.ShapeDtypeStruct((_N, _H1), jnp.float32),
    )(agg, agg, wcat)


def _z_kl_tc(agg2, eps2):
    def body(p0_ref, p1_ref, e_ref, z_ref, kl_ref):
        i = pl.program_id(0)
        both = p0_ref[...] + p1_ref[...]
        m = both[:, :_H2]
        ls = both[:, _H2:]
        z_ref[...] = m + jnp.exp(ls) * e_ref[...]
        klp = jnp.sum(1.0 + 2.0 * ls - m * m - jnp.exp(2.0 * ls))

        @pl.when(i == 0)
        def _():
            kl_ref[0, 0] = 0.0

        kl_ref[0, 0] += klp

    nb = _N // _TMA
    return pl.pallas_call(
        body,
        grid=(nb,),
        in_specs=[pl.BlockSpec((_TMA, _H1), lambda i: (i, 0)),
                  pl.BlockSpec((_TMA, _H1), lambda i: (i + nb, 0)),
                  pl.BlockSpec((_TMA, _H2), lambda i: (i, 0))],
        out_specs=[pl.BlockSpec((_TMA, _H2), lambda i: (i, 0)),
                   pl.BlockSpec(memory_space=pltpu.SMEM)],
        out_shape=[jax.ShapeDtypeStruct((_N, _H2), jnp.float32),
                   jax.ShapeDtypeStruct((1, 1), jnp.float32)],
    )(agg2, agg2, eps2)


_TMD = 200
_THRESH = 0.0016


def _decoder_tc(raw, z, kl_sum):
    nsteps = _N // _TMD

    def body(raw_ref, zr_ref, zf_ref, kl_ref, loss_ref,
             lab_acc, s1_acc, s2_acc):
        i = pl.program_id(0)

        @pl.when(i == 0)
        def _():
            lab_acc[0, 0] = 0.0
            s1_acc[0, 0] = 0.0
            s2_acc[0, 0] = 0.0

        hat = lax.dot_general(
            zr_ref[...], zf_ref[...], (((1,), (1,)), ((), ())),
            preferred_element_type=jnp.float32)
        lab = raw_ref[...] < _THRESH
        t = jnp.exp(-jnp.abs(hat))
        a = jnp.log(1.0 + t) + jnp.maximum(-hat, 0.0)
        s1_acc[0, 0] += jnp.sum(jnp.where(lab, a, 0.0))
        s2_acc[0, 0] += jnp.sum(jnp.where(lab, 0.0, a + hat))
        lab_acc[0, 0] += jnp.sum(jnp.where(lab, 1.0, 0.0))

        @pl.when(i == nsteps - 1)
        def _():
            adj_sum = lab_acc[0, 0]
            s1 = s1_acc[0, 0]
            s2 = s2_acc[0, 0]
            n2 = float(_N) * float(_N)
            norm = n2 / (2.0 * (n2 - adj_sum))
            pos_weight = (n2 - adj_sum) / adj_sum
            recon = norm * (pos_weight * s1 + s2) / n2
            kl = -0.5 * kl_ref[0, 0] / n2
            loss_ref[0, 0] = recon + kl

    return pl.pallas_call(
        body,
        grid=(nsteps,),
        in_specs=[pl.BlockSpec((_TMD, _N), lambda i: (i, 0)),
                  pl.BlockSpec((_TMD, _H2), lambda i: (i, 0)),
                  pl.BlockSpec((_N, _H2), lambda i: (0, 0)),
                  pl.BlockSpec(memory_space=pltpu.SMEM)],
        out_specs=pl.BlockSpec(memory_space=pltpu.SMEM),
        out_shape=jax.ShapeDtypeStruct((1, 1), jnp.float32),
        scratch_shapes=[pltpu.SMEM((1, 1), jnp.float32),
                        pltpu.SMEM((1, 1), jnp.float32),
                        pltpu.SMEM((1, 1), jnp.float32)],
    )(raw, z, z, kl_sum)


def kernel(features_norm, edge_index, edge_weight, eps, adj_label_raw,
           W1, W_mu, W_sigma):
    x = jnp.squeeze(features_norm, 0)
    eps2 = jnp.squeeze(eps, 0)

    pad = _EPAD - _E
    src3 = jnp.concatenate(
        [edge_index[0], jnp.zeros((pad,), jnp.int32)]).reshape(_NW, _G, _CH)
    dst3 = jnp.concatenate(
        [edge_index[1], jnp.zeros((pad,), jnp.int32)]).reshape(_NW, _G, _CH)
    ew3 = jnp.concatenate(
        [edge_weight, jnp.zeros((pad,), jnp.float32)]).reshape(_NW, _G, _CH)
    zeros_tile = jnp.zeros((_RPT, _H1), jnp.float32)

    support1 = _support1_tc(x, W1)
    agg1 = _spmm_sc(support1, src3, dst3, ew3, zeros_tile)
    sup2 = _layer2_tc(agg1, jnp.concatenate([W_mu, W_sigma], axis=1))
    agg2 = _spmm_sc(sup2, src3, dst3, ew3, zeros_tile)
    z, kl_sum = _z_kl_tc(agg2, eps2)
    loss = _decoder_tc(adj_label_raw, z, kl_sum)
    return loss[0, 0]
